# baseline (speedup 1.0000x reference)
import functools

import jax
import jax.numpy as jnp
from jax import lax
from jax.experimental import pallas as pl
from jax.experimental.pallas import tpu as pltpu
from jax.experimental.pallas import tpu_sc as plsc

B, Q, C = 1024, 900, 91
NW = 32
RPW = B // NW
NG = (Q + 15) // 16
LEN_OBJ = ((Q + 4) + 7) // 8 * 8


def _sc_matcher(class_logits, obj_flat, tgt):
    mesh = plsc.VectorSubcoreMesh(core_axis_name="c", subcore_axis_name="s")

    @functools.partial(
        pl.kernel,
        mesh=mesh,
        compiler_params=pltpu.CompilerParams(
            needs_layout_passes=False, use_tc_tiling_on_sc=True
        ),
        out_type=jax.ShapeDtypeStruct((B,), jnp.int32),
        scratch_types=[
            pltpu.VMEM((Q, C), jnp.float32),
            pltpu.VMEM((LEN_OBJ,), jnp.float32),
            pltpu.VMEM((RPW,), jnp.int32),
            pltpu.VMEM((RPW,), jnp.int32),
        ],
    )
    def k(cls_hbm, obj_hbm, tgt_hbm, out_hbm, cls_v, obj_v, tgt_v, res_v):
        wid = lax.axis_index("s") * 2 + lax.axis_index("c")
        lane = lax.iota(jnp.int32, 16)
        pltpu.sync_copy(tgt_hbm.at[pl.ds(wid * RPW, RPW)], tgt_v)

        def row_body(j, carry):
            b = wid * RPW + j
            pltpu.sync_copy(cls_hbm.at[b], cls_v)
            s_obj = b * Q
            a_obj = pl.multiple_of((s_obj // 8) * 8, 8)
            sh_obj = s_obj - a_obj
            pltpu.sync_copy(obj_hbm.at[pl.ds(a_obj, LEN_OBJ)], obj_v)
            t_vec = plsc.load_gather(tgt_v, [jnp.full((16,), j, jnp.int32)])

            def grp_body(g, gc):
                best_r, best_q = gc
                qv = jnp.minimum(g * 16, Q - 16) + lane
                a0 = jnp.zeros((16,), jnp.float32)
                a1 = jnp.zeros((16,), jnp.float32)
                a2 = jnp.zeros((16,), jnp.float32)
                a3 = jnp.zeros((16,), jnp.float32)
                cv = jnp.zeros((16,), jnp.int32)
                one = jnp.full((16,), 1, jnp.int32)
                for c in range(0, C - 3, 4):
                    a0 += jnp.exp(plsc.load_gather(cls_v, [qv, cv]))
                    cv1 = cv + one
                    a1 += jnp.exp(plsc.load_gather(cls_v, [qv, cv1]))
                    cv2 = cv1 + one
                    a2 += jnp.exp(plsc.load_gather(cls_v, [qv, cv2]))
                    cv3 = cv2 + one
                    a3 += jnp.exp(plsc.load_gather(cls_v, [qv, cv3]))
                    cv = cv3 + one
                for c in range(C - C % 4, C):
                    a0 += jnp.exp(plsc.load_gather(cls_v, [qv, cv]))
                    cv = cv + one
                ssum = (a0 + a1) + (a2 + a3)
                gv = plsc.load_gather(cls_v, [qv, t_vec])
                ov = plsc.load_gather(obj_v, [sh_obj + qv])
                r = jnp.exp(gv) / ((1.0 + jnp.exp(-ov)) * ssum)
                upd = r > best_r
                return jnp.where(upd, r, best_r), jnp.where(upd, qv, best_q)

            best_r, best_q = lax.fori_loop(
                0, NG, grp_body,
                (jnp.full((16,), -1.0, jnp.float32), jnp.zeros((16,), jnp.int32)),
            )
            m = jnp.max(best_r)
            cand = jnp.where(best_r == m, best_q, jnp.int32(2**30))
            res = jnp.broadcast_to(jnp.min(cand), (16,))
            plsc.store_scatter(
                res_v, [jnp.full((16,), j, jnp.int32)], res, mask=lane == 0
            )
            return carry

        lax.fori_loop(0, RPW, row_body, 0)
        pltpu.sync_copy(res_v, out_hbm.at[pl.ds(wid * RPW, RPW)])

    return k(class_logits, obj_flat, tgt)


def kernel(class_logits, obj_logits, targets):
    return _sc_matcher(
        class_logits, obj_logits.reshape(-1), targets.astype(jnp.int32)
    )


# speedup vs baseline: 5.6310x; 5.6310x over previous
import functools

import jax
import jax.numpy as jnp
from jax import lax
from jax.experimental import pallas as pl
from jax.experimental.pallas import tpu as pltpu
from jax.experimental.pallas import tpu_sc as plsc

B, Q, C = 1024, 900, 91
NQT = Q // 8
QTQ = NQT // 4


def _sc_matcher(cls_t, obj_t, tgt):
    mesh = plsc.VectorSubcoreMesh(core_axis_name="c", subcore_axis_name="s")

    @functools.partial(
        pl.kernel,
        mesh=mesh,
        compiler_params=pltpu.CompilerParams(needs_layout_passes=False),
        out_type=jax.ShapeDtypeStruct((B,), jnp.int32),
        scratch_types=[
            pltpu.VMEM((C, 8, 128), jnp.float32),
            pltpu.VMEM((8, 128), jnp.float32),
            pltpu.VMEM((128,), jnp.int32),
            pltpu.VMEM((128,), jnp.float32),
            pltpu.VMEM((128,), jnp.int32),
            pltpu.VMEM((128,), jnp.float32),
            pltpu.VMEM((128,), jnp.int32),
            pltpu.VMEM_SHARED((16, 128), jnp.float32),
            pltpu.VMEM_SHARED((16, 128), jnp.int32),
        ],
    )
    def k(cls_hbm, obj_hbm, tgt_hbm, out_hbm, cls_b, obj_b, tgt_v,
          br_v, bq_v, tr_v, tq_v, shr, shq):
        co = lax.axis_index("c")
        s = lax.axis_index("s")
        btl = lax.rem(s, 4)
        qq = s // 4
        b0 = (co * 4 + btl) * 128
        lane = lax.iota(jnp.int32, 16)
        pltpu.sync_copy(tgt_hbm.at[pl.ds(b0, 128)], tgt_v)

        def init_body(lg, carry):
            off = 16 * lg
            br_v[pl.ds(off, 16)] = jnp.full((16,), -1.0, jnp.float32)
            bq_v[pl.ds(off, 16)] = jnp.zeros((16,), jnp.int32)
            return carry

        lax.fori_loop(0, 8, init_body, 0)

        def chunk(q0, nqs):
            def body(i, carry):
                qs = i // 8
                lg = lax.rem(i, 8)
                off = 16 * lg
                br = br_v[pl.ds(off, 16)]
                bq = bq_v[pl.ds(off, 16)]
                a0 = jnp.zeros((16,), jnp.float32)
                a1 = jnp.zeros((16,), jnp.float32)
                a2 = jnp.zeros((16,), jnp.float32)
                a3 = jnp.zeros((16,), jnp.float32)
                for c in range(0, C - 3, 4):
                    a0 += jnp.exp(cls_b[c, qs, pl.ds(off, 16)])
                    a1 += jnp.exp(cls_b[c + 1, qs, pl.ds(off, 16)])
                    a2 += jnp.exp(cls_b[c + 2, qs, pl.ds(off, 16)])
                    a3 += jnp.exp(cls_b[c + 3, qs, pl.ds(off, 16)])
                for c in range(C - C % 4, C):
                    a0 += jnp.exp(cls_b[c, qs, pl.ds(off, 16)])
                ssum = (a0 + a1) + (a2 + a3)
                t16 = tgt_v[pl.ds(off, 16)]
                gv = plsc.load_gather(
                    cls_b,
                    [t16, jnp.full((16,), qs, jnp.int32), lane + off],
                )
                ov = obj_b[qs, pl.ds(off, 16)]
                r = jnp.exp(gv) / ((1.0 + jnp.exp(-ov)) * ssum)
                upd = r > br
                qvec = jnp.full((16,), q0 + qs, jnp.int32)
                br_v[pl.ds(off, 16)] = jnp.where(upd, r, br)
                bq_v[pl.ds(off, 16)] = jnp.where(upd, qvec, bq)
                return carry

            lax.fori_loop(0, nqs * 8, body, 0)

        def qt_body(qt, carry):
            q0 = qt * 8
            pltpu.sync_copy(
                cls_hbm.at[:, pl.ds(q0, 8), pl.ds(b0, 128)], cls_b
            )
            pltpu.sync_copy(obj_hbm.at[pl.ds(q0, 8), pl.ds(b0, 128)], obj_b)
            chunk(q0, 8)
            return carry

        lax.fori_loop(QTQ * qq, QTQ * qq + QTQ, qt_body, 0)

        pltpu.sync_copy(
            cls_hbm.at[:, pl.ds(8 * NQT, 4), pl.ds(b0, 128)],
            cls_b.at[:, pl.ds(0, 4), :],
        )
        pltpu.sync_copy(
            obj_hbm.at[pl.ds(8 * NQT, 4), pl.ds(b0, 128)],
            obj_b.at[pl.ds(0, 4), :],
        )
        chunk(8 * NQT, 4)

        pltpu.sync_copy(br_v, shr.at[s])
        pltpu.sync_copy(bq_v, shq.at[s])
        plsc.subcore_barrier()

        @pl.when(qq == 0)
        def _merge():
            for step in range(1, 4):
                pltpu.sync_copy(shr.at[4 * step + btl], tr_v)
                pltpu.sync_copy(shq.at[4 * step + btl], tq_v)

                def mbody(lg, carry):
                    off = 16 * lg
                    br = br_v[pl.ds(off, 16)]
                    bq = bq_v[pl.ds(off, 16)]
                    tr = tr_v[pl.ds(off, 16)]
                    tq = tq_v[pl.ds(off, 16)]
                    upd = (tr > br) | ((tr == br) & (tq < bq))
                    br_v[pl.ds(off, 16)] = jnp.where(upd, tr, br)
                    bq_v[pl.ds(off, 16)] = jnp.where(upd, tq, bq)
                    return carry

                lax.fori_loop(0, 8, mbody, 0)
            pltpu.sync_copy(bq_v, out_hbm.at[pl.ds(b0, 128)])

    return k(cls_t, obj_t, tgt)


def kernel(class_logits, obj_logits, targets):
    return _sc_matcher(
        jnp.transpose(class_logits, (2, 1, 0)),
        obj_logits.T,
        targets.astype(jnp.int32),
    )


# speedup vs baseline: 8.4661x; 1.5035x over previous
import functools

import jax
import jax.numpy as jnp
from jax import lax
from jax.experimental import pallas as pl
from jax.experimental.pallas import tpu as pltpu
from jax.experimental.pallas import tpu_sc as plsc

B, Q, C = 1024, 900, 91
CA = 46
CB = C - CA
NQT = Q // 8
QTQ = NQT // 4


def _sc_matcher(cls_t, obj_t, tgt):
    mesh = plsc.VectorSubcoreMesh(core_axis_name="c", subcore_axis_name="s")

    @functools.partial(
        pl.kernel,
        mesh=mesh,
        compiler_params=pltpu.CompilerParams(needs_layout_passes=False),
        out_type=jax.ShapeDtypeStruct((B,), jnp.int32),
        scratch_types=[
            pltpu.VMEM((CA, 8, 128), jnp.float32),
            pltpu.VMEM((CB, 8, 128), jnp.float32),
            pltpu.VMEM((8, 128), jnp.float32),
            pltpu.VMEM((8, 128), jnp.float32),
            pltpu.VMEM((8, 128), jnp.float32),
            pltpu.VMEM((128,), jnp.int32),
            pltpu.VMEM((128,), jnp.float32),
            pltpu.VMEM((128,), jnp.int32),
            pltpu.VMEM((128,), jnp.float32),
            pltpu.VMEM((128,), jnp.int32),
            pltpu.VMEM_SHARED((16, 128), jnp.float32),
            pltpu.VMEM_SHARED((16, 128), jnp.int32),
            pltpu.SemaphoreType.DMA,
            pltpu.SemaphoreType.DMA,
            pltpu.SemaphoreType.DMA,
        ],
    )
    def k(cls_hbm, obj_hbm, tgt_hbm, out_hbm, cls_a, cls_b, obj_b,
          sa_v, ga_v, tgt_v, br_v, bq_v, tr_v, tq_v, shr, shq,
          sem_a, sem_b, sem_o):
        co = lax.axis_index("c")
        s = lax.axis_index("s")
        btl = lax.rem(s, 4)
        qq = s // 4
        b0 = (co * 4 + btl) * 128
        lane = lax.iota(jnp.int32, 16)
        pltpu.sync_copy(tgt_hbm.at[pl.ds(b0, 128)], tgt_v)

        def init_body(lg, carry):
            off = 16 * lg
            br_v[pl.ds(off, 16)] = jnp.full((16,), -1.0, jnp.float32)
            bq_v[pl.ds(off, 16)] = jnp.zeros((16,), jnp.int32)
            return carry

        lax.fori_loop(0, 8, init_body, 0)

        def start_a(q0):
            pltpu.async_copy(
                cls_hbm.at[pl.ds(0, CA), pl.ds(q0, 8), pl.ds(b0, 128)],
                cls_a, sem_a,
            )

        def start_b(q0):
            pltpu.async_copy(
                cls_hbm.at[pl.ds(CA, CB), pl.ds(q0, 8), pl.ds(b0, 128)],
                cls_b, sem_b,
            )

        def wait_a():
            pltpu.make_async_copy(
                cls_hbm.at[pl.ds(0, CA), pl.ds(0, 8), pl.ds(0, 128)],
                cls_a, sem_a,
            ).wait()

        def wait_b():
            pltpu.make_async_copy(
                cls_hbm.at[pl.ds(CA, CB), pl.ds(0, 8), pl.ds(0, 128)],
                cls_b, sem_b,
            ).wait()

        def start_obj(q0):
            pltpu.async_copy(
                obj_hbm.at[pl.ds(q0, 8), pl.ds(b0, 128)], obj_b, sem_o
            )

        def wait_obj():
            pltpu.make_async_copy(
                obj_hbm.at[pl.ds(0, 8), pl.ds(0, 128)], obj_b, sem_o
            ).wait()

        def phase_a(nqs):
            def body(i, carry):
                qs = i // 8
                lg = lax.rem(i, 8)
                off = 16 * lg
                a0 = jnp.zeros((16,), jnp.float32)
                a1 = jnp.zeros((16,), jnp.float32)
                a2 = jnp.zeros((16,), jnp.float32)
                a3 = jnp.zeros((16,), jnp.float32)
                for c in range(0, CA - 3, 4):
                    a0 += jnp.exp(cls_a[c, qs, pl.ds(off, 16)])
                    a1 += jnp.exp(cls_a[c + 1, qs, pl.ds(off, 16)])
                    a2 += jnp.exp(cls_a[c + 2, qs, pl.ds(off, 16)])
                    a3 += jnp.exp(cls_a[c + 3, qs, pl.ds(off, 16)])
                for c in range(CA - CA % 4, CA):
                    a0 += jnp.exp(cls_a[c, qs, pl.ds(off, 16)])
                sa_v[qs, pl.ds(off, 16)] = (a0 + a1) + (a2 + a3)
                t16 = tgt_v[pl.ds(off, 16)]
                ta = jnp.where(t16 < CA, t16, CA - 1)
                ga_v[qs, pl.ds(off, 16)] = plsc.load_gather(
                    cls_a,
                    [ta, jnp.full((16,), qs, jnp.int32), lane + off],
                )
                return carry

            lax.fori_loop(0, nqs * 8, body, 0)

        def phase_b(q0, nqs):
            def body(i, carry):
                qs = i // 8
                lg = lax.rem(i, 8)
                off = 16 * lg
                br = br_v[pl.ds(off, 16)]
                bq = bq_v[pl.ds(off, 16)]
                a0 = jnp.zeros((16,), jnp.float32)
                a1 = jnp.zeros((16,), jnp.float32)
                a2 = jnp.zeros((16,), jnp.float32)
                a3 = jnp.zeros((16,), jnp.float32)
                for c in range(0, CB - 3, 4):
                    a0 += jnp.exp(cls_b[c, qs, pl.ds(off, 16)])
                    a1 += jnp.exp(cls_b[c + 1, qs, pl.ds(off, 16)])
                    a2 += jnp.exp(cls_b[c + 2, qs, pl.ds(off, 16)])
                    a3 += jnp.exp(cls_b[c + 3, qs, pl.ds(off, 16)])
                for c in range(CB - CB % 4, CB):
                    a0 += jnp.exp(cls_b[c, qs, pl.ds(off, 16)])
                ssum = sa_v[qs, pl.ds(off, 16)] + (a0 + a1) + (a2 + a3)
                t16 = tgt_v[pl.ds(off, 16)]
                tb = jnp.where(t16 < CA, 0, t16 - CA)
                gb = plsc.load_gather(
                    cls_b,
                    [tb, jnp.full((16,), qs, jnp.int32), lane + off],
                )
                gv = jnp.where(t16 < CA, ga_v[qs, pl.ds(off, 16)], gb)
                ov = obj_b[qs, pl.ds(off, 16)]
                r = jnp.exp(gv) / ((1.0 + jnp.exp(-ov)) * ssum)
                upd = r > br
                qvec = jnp.full((16,), q0 + qs, jnp.int32)
                br_v[pl.ds(off, 16)] = jnp.where(upd, r, br)
                bq_v[pl.ds(off, 16)] = jnp.where(upd, qvec, bq)
                return carry

            lax.fori_loop(0, nqs * 8, body, 0)

        qt_lo = QTQ * qq
        qt_hi = qt_lo + QTQ
        start_a(qt_lo * 8)

        def qt_body(qt, carry):
            q0 = qt * 8
            wait_a()
            start_b(q0)
            start_obj(q0)
            phase_a(8)
            wait_b()
            wait_obj()
            q_next = jnp.where(qt + 1 < qt_hi, (qt + 1) * 8, q0)
            start_a(q_next)
            phase_b(q0, 8)
            return carry

        lax.fori_loop(qt_lo, qt_hi, qt_body, 0)
        wait_a()

        pltpu.sync_copy(
            cls_hbm.at[pl.ds(0, CA), pl.ds(8 * NQT, 4), pl.ds(b0, 128)],
            cls_a.at[:, pl.ds(0, 4), :],
        )
        pltpu.sync_copy(
            cls_hbm.at[pl.ds(CA, CB), pl.ds(8 * NQT, 4), pl.ds(b0, 128)],
            cls_b.at[:, pl.ds(0, 4), :],
        )
        pltpu.sync_copy(
            obj_hbm.at[pl.ds(8 * NQT, 4), pl.ds(b0, 128)],
            obj_b.at[pl.ds(0, 4), :],
        )
        phase_a(4)
        phase_b(8 * NQT, 4)

        pltpu.sync_copy(br_v, shr.at[s])
        pltpu.sync_copy(bq_v, shq.at[s])
        plsc.subcore_barrier()

        @pl.when(qq == 0)
        def _merge():
            for step in range(1, 4):
                pltpu.sync_copy(shr.at[4 * step + btl], tr_v)
                pltpu.sync_copy(shq.at[4 * step + btl], tq_v)

                def mbody(lg, carry):
                    off = 16 * lg
                    br = br_v[pl.ds(off, 16)]
                    bq = bq_v[pl.ds(off, 16)]
                    tr = tr_v[pl.ds(off, 16)]
                    tq = tq_v[pl.ds(off, 16)]
                    upd = (tr > br) | ((tr == br) & (tq < bq))
                    br_v[pl.ds(off, 16)] = jnp.where(upd, tr, br)
                    bq_v[pl.ds(off, 16)] = jnp.where(upd, tq, bq)
                    return carry

                lax.fori_loop(0, 8, mbody, 0)
            pltpu.sync_copy(bq_v, out_hbm.at[pl.ds(b0, 128)])

    return k(cls_t, obj_t, tgt)


def kernel(class_logits, obj_logits, targets):
    return _sc_matcher(
        jnp.transpose(class_logits, (2, 1, 0)),
        obj_logits.T,
        targets.astype(jnp.int32),
    )
